# baseline (speedup 1.0000x reference)
import functools

import jax
import jax.numpy as jnp
from jax import lax
from jax.experimental import pallas as pl
from jax.experimental.pallas import tpu as pltpu
from jax.experimental.pallas import tpu_sc as plsc

N = 10000
E = 320000
D = 128
NLAYERS = 3
K = 8000
EPS = 1e-5

NC, NS = 2, 16
NW = NC * NS
EPW = 10240
EPAD = NW * EPW
BE = 128
STEPS = EPW // BE
CHUNK = 16
NCH = STEPS // CHUNK
NPAD = 10240
NR = NPAD // BE
RPT = NPAD // NS
RBLK = 512
NB = NPAD // RBLK
FB = 128
NFB = NPAD // FB

_HIGH = lax.Precision.HIGHEST


def _sc_rowsum_body(x_hbm, src_hbm, dst_hbm, out_hbm,
                    srcv, dstv, rows0, rows1, aggsh,
                    gsem0, gsem1, ssem0, ssem1):
    cid = lax.axis_index("c")
    sid = lax.axis_index("s")
    wid = sid * NC + cid

    def _z(i, _):
        r = i // 8
        c = (i % 8) * 16
        rows0[r, pl.ds(c, 16)] = jnp.zeros((16,), jnp.float32)
        return 0
    lax.fori_loop(0, BE * 8, _z, 0)

    def _zs(t, _):
        pltpu.sync_copy(rows0, aggsh.at[pl.ds(sid * RPT + t * BE, BE)])
        return 0
    lax.fori_loop(0, RPT // BE, _zs, 0)
    plsc.subcore_barrier()

    def _chunk(ci, _):
        base = wid * STEPS + ci * CHUNK
        pltpu.sync_copy(src_hbm.at[pl.ds(base, CHUNK)], srcv)
        pltpu.sync_copy(dst_hbm.at[pl.ds(base, CHUNK)], dstv)

        pltpu.async_copy(x_hbm.at[srcv.at[0]], rows0, gsem0)

        def _step(g, _):
            j0 = 2 * g
            j1 = 2 * g + 1
            pltpu.make_async_copy(x_hbm.at[srcv.at[j0]], rows0, gsem0).wait()

            @pl.when(g >= 1)
            def _w1():
                pltpu.make_async_copy(rows1, aggsh.at[dstv.at[j0 - 1]],
                                      ssem1).wait()

            pltpu.async_copy(x_hbm.at[srcv.at[j1]], rows1, gsem1)
            pltpu.async_copy(rows0, aggsh.at[dstv.at[j0]], ssem0, add=True)

            pltpu.make_async_copy(x_hbm.at[srcv.at[j1]], rows1, gsem1).wait()

            @pl.when(g < CHUNK // 2 - 1)
            def _g2():
                pltpu.make_async_copy(rows0, aggsh.at[dstv.at[j0]],
                                      ssem0).wait()
                pltpu.async_copy(x_hbm.at[srcv.at[j1 + 1]], rows0, gsem0)

            pltpu.async_copy(rows1, aggsh.at[dstv.at[j1]], ssem1, add=True)
            return 0

        lax.fori_loop(0, CHUNK // 2, _step, 0)

        pltpu.make_async_copy(rows0, aggsh.at[dstv.at[CHUNK - 2]],
                              ssem0).wait()
        pltpu.make_async_copy(rows1, aggsh.at[dstv.at[CHUNK - 1]],
                              ssem1).wait()
        return 0

    lax.fori_loop(0, NCH, _chunk, 0)
    plsc.subcore_barrier()

    pltpu.sync_copy(aggsh.at[pl.ds(sid * RPT, RPT)],
                    out_hbm.at[cid, pl.ds(sid * RPT, RPT)])


def _sc_rowsum(x_pad, src2d, dst2d):
    mesh = plsc.VectorSubcoreMesh(core_axis_name="c", subcore_axis_name="s")
    f = pl.kernel(
        _sc_rowsum_body,
        out_type=jax.ShapeDtypeStruct((NC, NPAD, D), jnp.float32),
        mesh=mesh,
        compiler_params=pltpu.CompilerParams(needs_layout_passes=False),
        scratch_types=[
            pltpu.VMEM((CHUNK, BE), jnp.int32),
            pltpu.VMEM((CHUNK, BE), jnp.int32),
            pltpu.VMEM((BE, D), jnp.float32),
            pltpu.VMEM((BE, D), jnp.float32),
            pltpu.VMEM_SHARED((NPAD, D), jnp.float32),
            pltpu.SemaphoreType.DMA,
            pltpu.SemaphoreType.DMA,
            pltpu.SemaphoreType.DMA,
            pltpu.SemaphoreType.DMA,
        ],
    )
    return f(x_pad, src2d, dst2d)


def _zero2d(ref, nrows):
    def _z(i, _):
        r = i // 8
        c = (i % 8) * 16
        ref[r, pl.ds(c, 16)] = jnp.zeros((16,), jnp.float32)
        return 0
    lax.fori_loop(0, nrows * 8, _z, 0)


RED_CH = 8
NRED = NR // RED_CH


def _sc_scalar_common(acc, total, tmpv, accsh, out_hbm, cid, sid):
    pltpu.sync_copy(acc, accsh.at[sid])
    plsc.subcore_barrier()

    @pl.when(sid < NRED)
    def _reduce():
        _zero2d(total, RED_CH)

        def _red(t, _):
            pltpu.sync_copy(accsh.at[t, pl.ds(sid * RED_CH, RED_CH)], tmpv)

            def _add(q, _):
                r = q // 8
                c = (q % 8) * 16
                total[r, pl.ds(c, 16)] = (total[r, pl.ds(c, 16)]
                                          + tmpv[r, pl.ds(c, 16)])
                return 0
            lax.fori_loop(0, RED_CH * 8, _add, 0)
            return 0
        lax.fori_loop(0, NS, _red, 0)

        pltpu.sync_copy(total, out_hbm.at[cid, pl.ds(sid * RED_CH, RED_CH)])


def _sc_count_body(dst_hbm, out_hbm, dstv, acc, total, tmpv, accsh):
    cid = lax.axis_index("c")
    sid = lax.axis_index("s")
    wid = sid * NC + cid

    _zero2d(acc, NR)

    pltpu.sync_copy(dst_hbm.at[pl.ds(wid * STEPS, STEPS)], dstv)

    ones = jnp.full((16,), 1.0, jnp.float32)

    def _cnt(i, _):
        r = i // 8
        c = (i % 8) * 16
        d16 = dstv[r, pl.ds(c, 16)]
        plsc.addupdate_scatter(acc, [d16 >> 7, d16 & 127], ones)
        return 0
    lax.fori_loop(0, EPW // 16, _cnt, 0)

    _sc_scalar_common(acc, total, tmpv, accsh, out_hbm, cid, sid)


def _sc_count(dst2d):
    mesh = plsc.VectorSubcoreMesh(core_axis_name="c", subcore_axis_name="s")
    f = pl.kernel(
        _sc_count_body,
        out_type=jax.ShapeDtypeStruct((NC, NR, BE), jnp.float32),
        mesh=mesh,
        compiler_params=pltpu.CompilerParams(needs_layout_passes=False),
        scratch_types=[
            pltpu.VMEM((STEPS, BE), jnp.int32),
            pltpu.VMEM((NR, BE), jnp.float32),
            pltpu.VMEM((RED_CH, BE), jnp.float32),
            pltpu.VMEM((RED_CH, BE), jnp.float32),
            pltpu.VMEM_SHARED((NS, NR, BE), jnp.float32),
        ],
    )
    return f(dst2d)


def _sc_segscalar_body(u_hbm, src_hbm, dst_hbm, out_hbm,
                       uv, srcv, dstv, acc, total, tmpv, accsh):
    cid = lax.axis_index("c")
    sid = lax.axis_index("s")
    wid = sid * NC + cid

    _zero2d(acc, NR)

    pltpu.sync_copy(u_hbm, uv)
    pltpu.sync_copy(src_hbm.at[pl.ds(wid * STEPS, STEPS)], srcv)
    pltpu.sync_copy(dst_hbm.at[pl.ds(wid * STEPS, STEPS)], dstv)

    def _seg(i, _):
        r = i // 8
        c = (i % 8) * 16
        s16 = srcv[r, pl.ds(c, 16)]
        d16 = dstv[r, pl.ds(c, 16)]
        vals = plsc.load_gather(uv, [s16 >> 7, s16 & 127])
        plsc.addupdate_scatter(acc, [d16 >> 7, d16 & 127], vals)
        return 0
    lax.fori_loop(0, EPW // 16, _seg, 0)

    _sc_scalar_common(acc, total, tmpv, accsh, out_hbm, cid, sid)


def _sc_segscalar(u2d, src2d, dst2d):
    mesh = plsc.VectorSubcoreMesh(core_axis_name="c", subcore_axis_name="s")
    f = pl.kernel(
        _sc_segscalar_body,
        out_type=jax.ShapeDtypeStruct((NC, NR, BE), jnp.float32),
        mesh=mesh,
        compiler_params=pltpu.CompilerParams(needs_layout_passes=False),
        scratch_types=[
            pltpu.VMEM((NR, BE), jnp.float32),
            pltpu.VMEM((STEPS, BE), jnp.int32),
            pltpu.VMEM((STEPS, BE), jnp.int32),
            pltpu.VMEM((NR, BE), jnp.float32),
            pltpu.VMEM((RED_CH, BE), jnp.float32),
            pltpu.VMEM((RED_CH, BE), jnp.float32),
            pltpu.VMEM_SHARED((NS, NR, BE), jnp.float32),
        ],
    )
    return f(u2d, src2d, dst2d)


def _tc_dense1_body(p0_ref, p1_ref, cnt_ref, x_ref, wl_ref, bl_ref, wr_ref,
                    y_ref, stats_ref):
    i = pl.program_id(0)

    @pl.when(i == 0)
    def _init():
        stats_ref[...] = jnp.zeros_like(stats_ref)

    agg = p0_ref[...] + p1_ref[...]
    cnt = cnt_ref[...]
    mean = agg / jnp.maximum(cnt, 1.0)
    y = (lax.dot_general(mean, wl_ref[...], (((1,), (1,)), ((), ())),
                         precision=_HIGH, preferred_element_type=jnp.float32)
         + bl_ref[...]
         + lax.dot_general(x_ref[...], wr_ref[...], (((1,), (1,)), ((), ())),
                           precision=_HIGH, preferred_element_type=jnp.float32))
    rows = i * RBLK + lax.broadcasted_iota(jnp.int32, (RBLK, 1), 0)
    y = jnp.where(rows < N, y, 0.0)
    y_ref[...] = y
    stats_ref[0:1, :] += jnp.sum(y, axis=0, keepdims=True)
    stats_ref[1:2, :] += jnp.sum(y * y, axis=0, keepdims=True)


def _tc_dense1(p0, p1, cnt_col, x, Wl, bl, Wr):
    return pl.pallas_call(
        _tc_dense1_body,
        grid=(NB,),
        in_specs=[
            pl.BlockSpec((RBLK, D), lambda i: (i, 0)),
            pl.BlockSpec((RBLK, D), lambda i: (i, 0)),
            pl.BlockSpec((RBLK, 1), lambda i: (i, 0)),
            pl.BlockSpec((RBLK, D), lambda i: (i, 0)),
            pl.BlockSpec((D, D), lambda i: (0, 0)),
            pl.BlockSpec((1, D), lambda i: (0, 0)),
            pl.BlockSpec((D, D), lambda i: (0, 0)),
        ],
        out_specs=[
            pl.BlockSpec((RBLK, D), lambda i: (i, 0)),
            pl.BlockSpec((2, D), lambda i: (0, 0)),
        ],
        out_shape=[
            jax.ShapeDtypeStruct((NPAD, D), jnp.float32),
            jax.ShapeDtypeStruct((2, D), jnp.float32),
        ],
    )(p0, p1, cnt_col, x, Wl, bl.reshape(1, D), Wr)


def _tc_dense2_body(y_ref, stats_ref, lnw_ref, lnb_ref, h_ref):
    i = pl.program_id(0)
    total = float(N * D)
    mu = jnp.sum(stats_ref[0:1, :]) / total
    var = jnp.sum(stats_ref[1:2, :]) / total - mu * mu
    inv = lax.rsqrt(var + EPS)
    z = (y_ref[...] - mu) * inv * lnw_ref[...] + lnb_ref[...]
    z = jnp.maximum(z, 0.0)
    rows = i * RBLK + lax.broadcasted_iota(jnp.int32, (RBLK, 1), 0)
    h_ref[...] = jnp.where(rows < N, z, 0.0)


def _tc_dense2(y, stats, lnw, lnb):
    return pl.pallas_call(
        _tc_dense2_body,
        grid=(NB,),
        in_specs=[
            pl.BlockSpec((RBLK, D), lambda i: (i, 0)),
            pl.BlockSpec((2, D), lambda i: (0, 0)),
            pl.BlockSpec((1, D), lambda i: (0, 0)),
            pl.BlockSpec((1, D), lambda i: (0, 0)),
        ],
        out_specs=pl.BlockSpec((RBLK, D), lambda i: (i, 0)),
        out_shape=jax.ShapeDtypeStruct((NPAD, D), jnp.float32),
    )(y, stats, lnw.reshape(1, D), lnb.reshape(1, D))


def _tc_score_body(h1_ref, h2_ref, h3_ref, wa1, wa2, wa3, wb1, wb2, wb3,
                   u_ref, r_ref):
    def proj(w1, w2, w3):
        return (lax.dot_general(h1_ref[...], w1[...], (((1,), (1,)), ((), ())),
                                precision=_HIGH,
                                preferred_element_type=jnp.float32)
                + lax.dot_general(h2_ref[...], w2[...],
                                  (((1,), (1,)), ((), ())), precision=_HIGH,
                                  preferred_element_type=jnp.float32)
                + lax.dot_general(h3_ref[...], w3[...],
                                  (((1,), (1,)), ((), ())), precision=_HIGH,
                                  preferred_element_type=jnp.float32))
    u_ref[...] = proj(wa1, wa2, wa3)
    r_ref[...] = proj(wb1, wb2, wb3)


def _tc_score(h1, h2, h3, Wrel, Wroot):
    wspecs = [pl.BlockSpec((1, D), lambda i: (0, 0))] * 6
    return pl.pallas_call(
        _tc_score_body,
        grid=(NB,),
        in_specs=[
            pl.BlockSpec((RBLK, D), lambda i: (i, 0)),
            pl.BlockSpec((RBLK, D), lambda i: (i, 0)),
            pl.BlockSpec((RBLK, D), lambda i: (i, 0)),
        ] + wspecs,
        out_specs=[
            pl.BlockSpec((RBLK, 1), lambda i: (i, 0)),
            pl.BlockSpec((RBLK, 1), lambda i: (i, 0)),
        ],
        out_shape=[
            jax.ShapeDtypeStruct((NPAD, 1), jnp.float32),
            jax.ShapeDtypeStruct((NPAD, 1), jnp.float32),
        ],
    )(h1, h2, h3,
      Wrel[:, 0:D], Wrel[:, D:2 * D], Wrel[:, 2 * D:3 * D],
      Wroot[:, 0:D], Wroot[:, D:2 * D], Wroot[:, 2 * D:3 * D])


def _tc_final_body(sa0_ref, sa1_ref, r_ref, brel_ref, h1_ref, h2_ref, h3_ref,
                   o1_ref, o2_ref, o3_ref, w_scr):
    b = pl.program_id(0)

    @pl.when(b == 0)
    def _thresh():
        score = sa0_ref[...] + sa1_ref[...] + r_ref[...] + brel_ref[0, 0]
        flat = (lax.broadcasted_iota(jnp.int32, (NFB, FB), 0) * FB
                + lax.broadcasted_iota(jnp.int32, (NFB, FB), 1))
        valid = flat < N
        bits = lax.bitcast_convert_type(score, jnp.uint32)
        neg = bits >> 31
        key = jnp.where(neg == 1, ~bits, bits | jnp.uint32(0x80000000))
        key = jnp.where(valid, key, jnp.uint32(0))

        def _bit(bi, t):
            cand = t | (jnp.uint32(1) << (31 - bi))
            c = jnp.sum((key >= cand).astype(jnp.float32))
            return jnp.where(c >= K, cand, t)
        tkey = lax.fori_loop(0, 32, _bit, jnp.uint32(0))

        c1 = jnp.sum((key > tkey).astype(jnp.float32))
        m = jnp.sum((key == tkey).astype(jnp.float32))
        frac = (K - c1) / jnp.maximum(m, 1.0)
        sel = jnp.where(key > tkey, 1.0, jnp.where(key == tkey, frac, 0.0))
        w_scr[...] = jnp.tanh(score) * sel
        o1_ref[...] = jnp.zeros_like(o1_ref)
        o2_ref[...] = jnp.zeros_like(o2_ref)
        o3_ref[...] = jnp.zeros_like(o3_ref)

    wrow = w_scr[pl.ds(b, 1), :]
    dn = (((1,), (0,)), ((), ()))
    o1_ref[...] += lax.dot_general(wrow, h1_ref[...], dn, precision=_HIGH,
                                   preferred_element_type=jnp.float32)
    o2_ref[...] += lax.dot_general(wrow, h2_ref[...], dn, precision=_HIGH,
                                   preferred_element_type=jnp.float32)
    o3_ref[...] += lax.dot_general(wrow, h3_ref[...], dn, precision=_HIGH,
                                   preferred_element_type=jnp.float32)


def _tc_final(sa0, sa1, r2d, brel, h1, h2, h3):
    return pl.pallas_call(
        _tc_final_body,
        grid=(NFB,),
        in_specs=[
            pl.BlockSpec((NFB, FB), lambda i: (0, 0)),
            pl.BlockSpec((NFB, FB), lambda i: (0, 0)),
            pl.BlockSpec((NFB, FB), lambda i: (0, 0)),
            pl.BlockSpec((1, 1), lambda i: (0, 0)),
            pl.BlockSpec((FB, D), lambda i: (i, 0)),
            pl.BlockSpec((FB, D), lambda i: (i, 0)),
            pl.BlockSpec((FB, D), lambda i: (i, 0)),
        ],
        out_specs=[
            pl.BlockSpec((1, D), lambda i: (0, 0)),
            pl.BlockSpec((1, D), lambda i: (0, 0)),
            pl.BlockSpec((1, D), lambda i: (0, 0)),
        ],
        out_shape=[
            jax.ShapeDtypeStruct((1, D), jnp.float32),
            jax.ShapeDtypeStruct((1, D), jnp.float32),
            jax.ShapeDtypeStruct((1, D), jnp.float32),
        ],
        scratch_shapes=[pltpu.VMEM((NFB, FB), jnp.float32)],
    )(sa0, sa1, r2d, brel, h1, h2, h3)


def kernel(x, edge_index, batch, Wl0, bl0, Wr0, lnw0, lnb0,
           Wl1, bl1, Wr1, lnw1, lnb1, Wl2, bl2, Wr2, lnw2, lnb2,
           Wrel, brel, Wroot):
    del batch

    src = edge_index[0]
    dst = edge_index[1]
    npad_e = EPAD - E
    src_flat = jnp.concatenate([src, jnp.zeros((npad_e,), jnp.int32)])
    dst_flat = jnp.concatenate([dst, jnp.full((npad_e,), N, jnp.int32)])
    src2d = src_flat.reshape(EPAD // BE, BE)
    dst2d = dst_flat.reshape(EPAD // BE, BE)

    xp = jnp.pad(x, ((0, NPAD - N), (0, 0)))

    cnt = _sc_count(dst2d)
    cnt_col = (cnt[0] + cnt[1]).reshape(NPAD, 1)

    params = [(Wl0, bl0, Wr0, lnw0, lnb0),
              (Wl1, bl1, Wr1, lnw1, lnb1),
              (Wl2, bl2, Wr2, lnw2, lnb2)]

    h = []
    cur = xp
    for (Wl, bl, Wr, lnw, lnb) in params:
        part = _sc_rowsum(cur, src2d, dst2d)
        y, stats = _tc_dense1(part[0], part[1], cnt_col, cur, Wl, bl, Wr)
        cur = _tc_dense2(y, stats, lnw, lnb)
        h.append(cur)

    h1, h2, h3 = h
    u, r = _tc_score(h1, h2, h3, Wrel, Wroot)
    sa = _sc_segscalar(u.reshape(NR, BE), src2d, dst2d)

    o1, o2, o3 = _tc_final(sa[0], sa[1],
                           r.reshape(NFB, FB), brel.reshape(1, 1),
                           h1, h2, h3)
    return jnp.concatenate([o1, o2, o3], axis=1)


# speedup vs baseline: 1.0081x; 1.0081x over previous
import functools

import jax
import jax.numpy as jnp
from jax import lax
from jax.experimental import pallas as pl
from jax.experimental.pallas import tpu as pltpu
from jax.experimental.pallas import tpu_sc as plsc

N = 10000
E = 320000
D = 128
NLAYERS = 3
K = 8000
EPS = 1e-5

NC, NS = 2, 16
NW = NC * NS
EPW = 10240
EPAD = NW * EPW
BE = 128
STEPS = EPW // BE
CHUNK = 40
NCH = STEPS // CHUNK
NPAD = 10240
NR = NPAD // BE
RPT = NPAD // NS
RBLK = 512
NB = NPAD // RBLK
FB = 128
NFB = NPAD // FB

_HIGH = lax.Precision.HIGHEST


def _sc_rowsum_body(x_hbm, src_hbm, dst_hbm, out_hbm,
                    srcv, dstv, rows0, rows1, aggsh,
                    gsem0, gsem1, ssem0, ssem1):
    cid = lax.axis_index("c")
    sid = lax.axis_index("s")
    wid = sid * NC + cid

    def _z(i, _):
        r = i // 8
        c = (i % 8) * 16
        rows0[r, pl.ds(c, 16)] = jnp.zeros((16,), jnp.float32)
        return 0
    lax.fori_loop(0, BE * 8, _z, 0)

    def _zs(t, _):
        pltpu.sync_copy(rows0, aggsh.at[pl.ds(sid * RPT + t * BE, BE)])
        return 0
    lax.fori_loop(0, RPT // BE, _zs, 0)
    plsc.subcore_barrier()

    def _chunk(ci, _):
        base = wid * STEPS + ci * CHUNK
        pltpu.sync_copy(src_hbm.at[pl.ds(base, CHUNK)], srcv)
        pltpu.sync_copy(dst_hbm.at[pl.ds(base, CHUNK)], dstv)

        pltpu.async_copy(x_hbm.at[srcv.at[0]], rows0, gsem0)

        def _step(g, _):
            j0 = 2 * g
            j1 = 2 * g + 1
            pltpu.make_async_copy(x_hbm.at[srcv.at[j0]], rows0, gsem0).wait()

            @pl.when(g >= 1)
            def _w1():
                pltpu.make_async_copy(rows1, aggsh.at[dstv.at[j0 - 1]],
                                      ssem1).wait()

            pltpu.async_copy(x_hbm.at[srcv.at[j1]], rows1, gsem1)
            pltpu.async_copy(rows0, aggsh.at[dstv.at[j0]], ssem0, add=True)

            pltpu.make_async_copy(x_hbm.at[srcv.at[j1]], rows1, gsem1).wait()

            @pl.when(g < CHUNK // 2 - 1)
            def _g2():
                pltpu.make_async_copy(rows0, aggsh.at[dstv.at[j0]],
                                      ssem0).wait()
                pltpu.async_copy(x_hbm.at[srcv.at[j1 + 1]], rows0, gsem0)

            pltpu.async_copy(rows1, aggsh.at[dstv.at[j1]], ssem1, add=True)
            return 0

        lax.fori_loop(0, CHUNK // 2, _step, 0)

        pltpu.make_async_copy(rows0, aggsh.at[dstv.at[CHUNK - 2]],
                              ssem0).wait()
        pltpu.make_async_copy(rows1, aggsh.at[dstv.at[CHUNK - 1]],
                              ssem1).wait()
        return 0

    lax.fori_loop(0, NCH, _chunk, 0)
    plsc.subcore_barrier()

    pltpu.sync_copy(aggsh.at[pl.ds(sid * RPT, RPT)],
                    out_hbm.at[cid, pl.ds(sid * RPT, RPT)])


def _sc_rowsum(x_pad, src2d, dst2d):
    mesh = plsc.VectorSubcoreMesh(core_axis_name="c", subcore_axis_name="s")
    f = pl.kernel(
        _sc_rowsum_body,
        out_type=jax.ShapeDtypeStruct((NC, NPAD, D), jnp.float32),
        mesh=mesh,
        compiler_params=pltpu.CompilerParams(needs_layout_passes=False),
        scratch_types=[
            pltpu.VMEM((CHUNK, BE), jnp.int32),
            pltpu.VMEM((CHUNK, BE), jnp.int32),
            pltpu.VMEM((BE, D), jnp.float32),
            pltpu.VMEM((BE, D), jnp.float32),
            pltpu.VMEM_SHARED((NPAD, D), jnp.float32),
            pltpu.SemaphoreType.DMA,
            pltpu.SemaphoreType.DMA,
            pltpu.SemaphoreType.DMA,
            pltpu.SemaphoreType.DMA,
        ],
    )
    return f(x_pad, src2d, dst2d)


def _zero2d(ref, nrows):
    def _z(i, _):
        r = i // 8
        c = (i % 8) * 16
        ref[r, pl.ds(c, 16)] = jnp.zeros((16,), jnp.float32)
        return 0
    lax.fori_loop(0, nrows * 8, _z, 0)


RED_CH = 8
NRED = NR // RED_CH


def _sc_scalar_common(acc, total, tmpv, accsh, out_hbm, cid, sid):
    pltpu.sync_copy(acc, accsh.at[sid])
    plsc.subcore_barrier()

    @pl.when(sid < NRED)
    def _reduce():
        _zero2d(total, RED_CH)

        def _red(t, _):
            pltpu.sync_copy(accsh.at[t, pl.ds(sid * RED_CH, RED_CH)], tmpv)

            def _add(q, _):
                r = q // 8
                c = (q % 8) * 16
                total[r, pl.ds(c, 16)] = (total[r, pl.ds(c, 16)]
                                          + tmpv[r, pl.ds(c, 16)])
                return 0
            lax.fori_loop(0, RED_CH * 8, _add, 0)
            return 0
        lax.fori_loop(0, NS, _red, 0)

        pltpu.sync_copy(total, out_hbm.at[cid, pl.ds(sid * RED_CH, RED_CH)])


def _sc_count_body(dst_hbm, out_hbm, dstv, acc, total, tmpv, accsh):
    cid = lax.axis_index("c")
    sid = lax.axis_index("s")
    wid = sid * NC + cid

    _zero2d(acc, NR)

    pltpu.sync_copy(dst_hbm.at[pl.ds(wid * STEPS, STEPS)], dstv)

    ones = jnp.full((16,), 1.0, jnp.float32)

    def _cnt(i, _):
        r = i // 8
        c = (i % 8) * 16
        d16 = dstv[r, pl.ds(c, 16)]
        plsc.addupdate_scatter(acc, [d16 >> 7, d16 & 127], ones)
        return 0
    lax.fori_loop(0, EPW // 16, _cnt, 0)

    _sc_scalar_common(acc, total, tmpv, accsh, out_hbm, cid, sid)


def _sc_count(dst2d):
    mesh = plsc.VectorSubcoreMesh(core_axis_name="c", subcore_axis_name="s")
    f = pl.kernel(
        _sc_count_body,
        out_type=jax.ShapeDtypeStruct((NC, NR, BE), jnp.float32),
        mesh=mesh,
        compiler_params=pltpu.CompilerParams(needs_layout_passes=False),
        scratch_types=[
            pltpu.VMEM((STEPS, BE), jnp.int32),
            pltpu.VMEM((NR, BE), jnp.float32),
            pltpu.VMEM((RED_CH, BE), jnp.float32),
            pltpu.VMEM((RED_CH, BE), jnp.float32),
            pltpu.VMEM_SHARED((NS, NR, BE), jnp.float32),
        ],
    )
    return f(dst2d)


def _sc_segscalar_body(u_hbm, src_hbm, dst_hbm, out_hbm,
                       uv, srcv, dstv, acc, total, tmpv, accsh):
    cid = lax.axis_index("c")
    sid = lax.axis_index("s")
    wid = sid * NC + cid

    _zero2d(acc, NR)

    pltpu.sync_copy(u_hbm, uv)
    pltpu.sync_copy(src_hbm.at[pl.ds(wid * STEPS, STEPS)], srcv)
    pltpu.sync_copy(dst_hbm.at[pl.ds(wid * STEPS, STEPS)], dstv)

    def _seg(i, _):
        r = i // 8
        c = (i % 8) * 16
        s16 = srcv[r, pl.ds(c, 16)]
        d16 = dstv[r, pl.ds(c, 16)]
        vals = plsc.load_gather(uv, [s16 >> 7, s16 & 127])
        plsc.addupdate_scatter(acc, [d16 >> 7, d16 & 127], vals)
        return 0
    lax.fori_loop(0, EPW // 16, _seg, 0)

    _sc_scalar_common(acc, total, tmpv, accsh, out_hbm, cid, sid)


def _sc_segscalar(u2d, src2d, dst2d):
    mesh = plsc.VectorSubcoreMesh(core_axis_name="c", subcore_axis_name="s")
    f = pl.kernel(
        _sc_segscalar_body,
        out_type=jax.ShapeDtypeStruct((NC, NR, BE), jnp.float32),
        mesh=mesh,
        compiler_params=pltpu.CompilerParams(needs_layout_passes=False),
        scratch_types=[
            pltpu.VMEM((NR, BE), jnp.float32),
            pltpu.VMEM((STEPS, BE), jnp.int32),
            pltpu.VMEM((STEPS, BE), jnp.int32),
            pltpu.VMEM((NR, BE), jnp.float32),
            pltpu.VMEM((RED_CH, BE), jnp.float32),
            pltpu.VMEM((RED_CH, BE), jnp.float32),
            pltpu.VMEM_SHARED((NS, NR, BE), jnp.float32),
        ],
    )
    return f(u2d, src2d, dst2d)


def _tc_dense1_body(p0_ref, p1_ref, cnt_ref, x_ref, wl_ref, bl_ref, wr_ref,
                    y_ref, stats_ref):
    i = pl.program_id(0)

    @pl.when(i == 0)
    def _init():
        stats_ref[...] = jnp.zeros_like(stats_ref)

    agg = p0_ref[...] + p1_ref[...]
    cnt = cnt_ref[...]
    mean = agg / jnp.maximum(cnt, 1.0)
    y = (lax.dot_general(mean, wl_ref[...], (((1,), (1,)), ((), ())),
                         precision=_HIGH, preferred_element_type=jnp.float32)
         + bl_ref[...]
         + lax.dot_general(x_ref[...], wr_ref[...], (((1,), (1,)), ((), ())),
                           precision=_HIGH, preferred_element_type=jnp.float32))
    rows = i * RBLK + lax.broadcasted_iota(jnp.int32, (RBLK, 1), 0)
    y = jnp.where(rows < N, y, 0.0)
    y_ref[...] = y
    stats_ref[0:1, :] += jnp.sum(y, axis=0, keepdims=True)
    stats_ref[1:2, :] += jnp.sum(y * y, axis=0, keepdims=True)


def _tc_dense1(p0, p1, cnt_col, x, Wl, bl, Wr):
    return pl.pallas_call(
        _tc_dense1_body,
        grid=(NB,),
        in_specs=[
            pl.BlockSpec((RBLK, D), lambda i: (i, 0)),
            pl.BlockSpec((RBLK, D), lambda i: (i, 0)),
            pl.BlockSpec((RBLK, 1), lambda i: (i, 0)),
            pl.BlockSpec((RBLK, D), lambda i: (i, 0)),
            pl.BlockSpec((D, D), lambda i: (0, 0)),
            pl.BlockSpec((1, D), lambda i: (0, 0)),
            pl.BlockSpec((D, D), lambda i: (0, 0)),
        ],
        out_specs=[
            pl.BlockSpec((RBLK, D), lambda i: (i, 0)),
            pl.BlockSpec((2, D), lambda i: (0, 0)),
        ],
        out_shape=[
            jax.ShapeDtypeStruct((NPAD, D), jnp.float32),
            jax.ShapeDtypeStruct((2, D), jnp.float32),
        ],
    )(p0, p1, cnt_col, x, Wl, bl.reshape(1, D), Wr)


def _tc_dense2_body(y_ref, stats_ref, lnw_ref, lnb_ref, h_ref):
    i = pl.program_id(0)
    total = float(N * D)
    mu = jnp.sum(stats_ref[0:1, :]) / total
    var = jnp.sum(stats_ref[1:2, :]) / total - mu * mu
    inv = lax.rsqrt(var + EPS)
    z = (y_ref[...] - mu) * inv * lnw_ref[...] + lnb_ref[...]
    z = jnp.maximum(z, 0.0)
    rows = i * RBLK + lax.broadcasted_iota(jnp.int32, (RBLK, 1), 0)
    h_ref[...] = jnp.where(rows < N, z, 0.0)


def _tc_dense2(y, stats, lnw, lnb):
    return pl.pallas_call(
        _tc_dense2_body,
        grid=(NB,),
        in_specs=[
            pl.BlockSpec((RBLK, D), lambda i: (i, 0)),
            pl.BlockSpec((2, D), lambda i: (0, 0)),
            pl.BlockSpec((1, D), lambda i: (0, 0)),
            pl.BlockSpec((1, D), lambda i: (0, 0)),
        ],
        out_specs=pl.BlockSpec((RBLK, D), lambda i: (i, 0)),
        out_shape=jax.ShapeDtypeStruct((NPAD, D), jnp.float32),
    )(y, stats, lnw.reshape(1, D), lnb.reshape(1, D))


def _tc_score_body(h1_ref, h2_ref, h3_ref, wa1, wa2, wa3, wb1, wb2, wb3,
                   u_ref, r_ref):
    def proj(w1, w2, w3):
        return (lax.dot_general(h1_ref[...], w1[...], (((1,), (1,)), ((), ())),
                                precision=_HIGH,
                                preferred_element_type=jnp.float32)
                + lax.dot_general(h2_ref[...], w2[...],
                                  (((1,), (1,)), ((), ())), precision=_HIGH,
                                  preferred_element_type=jnp.float32)
                + lax.dot_general(h3_ref[...], w3[...],
                                  (((1,), (1,)), ((), ())), precision=_HIGH,
                                  preferred_element_type=jnp.float32))
    u_ref[...] = proj(wa1, wa2, wa3)
    r_ref[...] = proj(wb1, wb2, wb3)


def _tc_score(h1, h2, h3, Wrel, Wroot):
    wspecs = [pl.BlockSpec((1, D), lambda i: (0, 0))] * 6
    return pl.pallas_call(
        _tc_score_body,
        grid=(NB,),
        in_specs=[
            pl.BlockSpec((RBLK, D), lambda i: (i, 0)),
            pl.BlockSpec((RBLK, D), lambda i: (i, 0)),
            pl.BlockSpec((RBLK, D), lambda i: (i, 0)),
        ] + wspecs,
        out_specs=[
            pl.BlockSpec((RBLK, 1), lambda i: (i, 0)),
            pl.BlockSpec((RBLK, 1), lambda i: (i, 0)),
        ],
        out_shape=[
            jax.ShapeDtypeStruct((NPAD, 1), jnp.float32),
            jax.ShapeDtypeStruct((NPAD, 1), jnp.float32),
        ],
    )(h1, h2, h3,
      Wrel[:, 0:D], Wrel[:, D:2 * D], Wrel[:, 2 * D:3 * D],
      Wroot[:, 0:D], Wroot[:, D:2 * D], Wroot[:, 2 * D:3 * D])


def _tc_final_body(sa0_ref, sa1_ref, r_ref, brel_ref, h1_ref, h2_ref, h3_ref,
                   o1_ref, o2_ref, o3_ref, w_scr):
    b = pl.program_id(0)

    @pl.when(b == 0)
    def _thresh():
        score = sa0_ref[...] + sa1_ref[...] + r_ref[...] + brel_ref[0, 0]
        flat = (lax.broadcasted_iota(jnp.int32, (NFB, FB), 0) * FB
                + lax.broadcasted_iota(jnp.int32, (NFB, FB), 1))
        valid = flat < N
        bits = lax.bitcast_convert_type(score, jnp.uint32)
        neg = bits >> 31
        key = jnp.where(neg == 1, ~bits, bits | jnp.uint32(0x80000000))
        key = jnp.where(valid, key, jnp.uint32(0))

        def _bit(bi, t):
            cand = t | (jnp.uint32(1) << (31 - bi))
            c = jnp.sum((key >= cand).astype(jnp.float32))
            return jnp.where(c >= K, cand, t)
        tkey = lax.fori_loop(0, 32, _bit, jnp.uint32(0))

        c1 = jnp.sum((key > tkey).astype(jnp.float32))
        m = jnp.sum((key == tkey).astype(jnp.float32))
        frac = (K - c1) / jnp.maximum(m, 1.0)
        sel = jnp.where(key > tkey, 1.0, jnp.where(key == tkey, frac, 0.0))
        w_scr[...] = jnp.tanh(score) * sel
        o1_ref[...] = jnp.zeros_like(o1_ref)
        o2_ref[...] = jnp.zeros_like(o2_ref)
        o3_ref[...] = jnp.zeros_like(o3_ref)

    wrow = w_scr[pl.ds(b, 1), :]
    dn = (((1,), (0,)), ((), ()))
    o1_ref[...] += lax.dot_general(wrow, h1_ref[...], dn, precision=_HIGH,
                                   preferred_element_type=jnp.float32)
    o2_ref[...] += lax.dot_general(wrow, h2_ref[...], dn, precision=_HIGH,
                                   preferred_element_type=jnp.float32)
    o3_ref[...] += lax.dot_general(wrow, h3_ref[...], dn, precision=_HIGH,
                                   preferred_element_type=jnp.float32)


def _tc_final(sa0, sa1, r2d, brel, h1, h2, h3):
    return pl.pallas_call(
        _tc_final_body,
        grid=(NFB,),
        in_specs=[
            pl.BlockSpec((NFB, FB), lambda i: (0, 0)),
            pl.BlockSpec((NFB, FB), lambda i: (0, 0)),
            pl.BlockSpec((NFB, FB), lambda i: (0, 0)),
            pl.BlockSpec((1, 1), lambda i: (0, 0)),
            pl.BlockSpec((FB, D), lambda i: (i, 0)),
            pl.BlockSpec((FB, D), lambda i: (i, 0)),
            pl.BlockSpec((FB, D), lambda i: (i, 0)),
        ],
        out_specs=[
            pl.BlockSpec((1, D), lambda i: (0, 0)),
            pl.BlockSpec((1, D), lambda i: (0, 0)),
            pl.BlockSpec((1, D), lambda i: (0, 0)),
        ],
        out_shape=[
            jax.ShapeDtypeStruct((1, D), jnp.float32),
            jax.ShapeDtypeStruct((1, D), jnp.float32),
            jax.ShapeDtypeStruct((1, D), jnp.float32),
        ],
        scratch_shapes=[pltpu.VMEM((NFB, FB), jnp.float32)],
    )(sa0, sa1, r2d, brel, h1, h2, h3)


def kernel(x, edge_index, batch, Wl0, bl0, Wr0, lnw0, lnb0,
           Wl1, bl1, Wr1, lnw1, lnb1, Wl2, bl2, Wr2, lnw2, lnb2,
           Wrel, brel, Wroot):
    del batch

    src = edge_index[0]
    dst = edge_index[1]
    npad_e = EPAD - E
    src_flat = jnp.concatenate([src, jnp.zeros((npad_e,), jnp.int32)])
    dst_flat = jnp.concatenate([dst, jnp.full((npad_e,), N, jnp.int32)])
    src2d = src_flat.reshape(EPAD // BE, BE)
    dst2d = dst_flat.reshape(EPAD // BE, BE)

    xp = jnp.pad(x, ((0, NPAD - N), (0, 0)))

    cnt = _sc_count(dst2d)
    cnt_col = (cnt[0] + cnt[1]).reshape(NPAD, 1)

    params = [(Wl0, bl0, Wr0, lnw0, lnb0),
              (Wl1, bl1, Wr1, lnw1, lnb1),
              (Wl2, bl2, Wr2, lnw2, lnb2)]

    h = []
    cur = xp
    for (Wl, bl, Wr, lnw, lnb) in params:
        part = _sc_rowsum(cur, src2d, dst2d)
        y, stats = _tc_dense1(part[0], part[1], cnt_col, cur, Wl, bl, Wr)
        cur = _tc_dense2(y, stats, lnw, lnb)
        h.append(cur)

    h1, h2, h3 = h
    u, r = _tc_score(h1, h2, h3, Wrel, Wroot)
    sa = _sc_segscalar(u.reshape(NR, BE), src2d, dst2d)

    o1, o2, o3 = _tc_final(sa[0], sa[1],
                           r.reshape(NFB, FB), brel.reshape(1, 1),
                           h1, h2, h3)
    return jnp.concatenate([o1, o2, o3], axis=1)
